# baseline (speedup 1.0000x reference)
import functools

import jax
import jax.numpy as jnp
from jax import lax
from jax.experimental import pallas as pl
from jax.experimental.pallas import tpu as pltpu
from jax.experimental.pallas import tpu_sc as plsc

EMBED_DIM = 32
PACK = 4
CHUNK = 128
NBUF = 4
GROUPS = CHUNK // 16


@functools.cache
def _make_kernel(n_flat: int, vocab: int):
    info = plsc.get_sparse_core_info()
    num_workers = info.num_cores * info.num_subcores
    b_per_w = n_flat // num_workers
    steps = b_per_w // CHUNK
    mesh = plsc.VectorSubcoreMesh(core_axis_name="c", subcore_axis_name="s")

    @functools.partial(
        pl.kernel,
        mesh=mesh,
        out_type=jax.ShapeDtypeStruct((n_flat * EMBED_DIM,), jnp.float32),
        scratch_types=[
            pltpu.VMEM((NBUF, CHUNK), jnp.int32),
            pltpu.VMEM((NBUF, CHUNK), jnp.int32),
            *[pltpu.VMEM((CHUNK, 128), jnp.float32) for _ in range(NBUF)],
            *[pltpu.VMEM((CHUNK * EMBED_DIM,), jnp.float32) for _ in range(NBUF)],
            pltpu.SemaphoreType.DMA,
            pltpu.SemaphoreType.DMA,
            pltpu.SemaphoreType.DMA,
        ],
        compiler_params=pltpu.CompilerParams(needs_layout_passes=False),
    )
    def emb_kernel(idx_hbm, table_hbm, out_hbm, ibuf, dbuf, *rest):
        gbufs = rest[:NBUF]
        obufs = rest[NBUF : 2 * NBUF]
        isem, gsem, osem = rest[2 * NBUF : 2 * NBUF + 3]
        wid = lax.axis_index("s") * info.num_cores + lax.axis_index("c")
        base = wid * b_per_w
        my_idx = idx_hbm.at[wid]

        iota = lax.iota(jnp.int32, 16)

        def fetch_idx(j, slot):
            pltpu.async_copy(my_idx.at[j], ibuf.at[slot], isem)

        def wait_idx(slot):
            pltpu.make_async_copy(my_idx.at[0], ibuf.at[slot], isem).wait()

        def div_and_gather(j, slot):
            for g in range(GROUPS):
                iv = ibuf[slot, pl.ds(g * 16, 16)]
                dbuf[slot, pl.ds(g * 16, 16)] = lax.shift_right_logical(iv, 2)
            pltpu.async_copy(table_hbm.at[dbuf.at[slot]], gbufs[slot], gsem)

        def wait_gather(slot):
            pltpu.make_async_copy(
                table_hbm.at[dbuf.at[slot]], gbufs[slot], gsem
            ).wait()

        def extract(slot):
            for g in range(GROUPS):
                rows = g * 16 + iota
                iv = ibuf[slot, pl.ds(g * 16, 16)]
                colbase = lax.shift_left(jnp.bitwise_and(iv, PACK - 1), 5)
                obase = rows * EMBED_DIM
                for c in range(EMBED_DIM):
                    val = plsc.load_gather(gbufs[slot], [rows, colbase + c])
                    plsc.store_scatter(obufs[slot], [obase + c], val)

        def fire_out(j, slot):
            pltpu.async_copy(
                obufs[slot],
                out_hbm.at[pl.ds((base + j * CHUNK) * EMBED_DIM, CHUNK * EMBED_DIM)],
                osem,
            )

        def drain_out(slot):
            pltpu.make_async_copy(
                obufs[slot],
                out_hbm.at[pl.ds(0, CHUNK * EMBED_DIM)],
                osem,
            ).wait()

        fetch_idx(0, 0)
        fetch_idx(1, 1)
        wait_idx(0)
        div_and_gather(0, 0)

        def outer(jo, _):
            for b in range(NBUF):
                j = jo * NBUF + b
                b1 = (b + 1) % NBUF
                b2 = (b + 2) % NBUF

                @pl.when(j + 2 < steps)
                def _fetch():
                    fetch_idx(j + 2, b2)

                @pl.when(j + 1 < steps)
                def _gather():
                    wait_idx(b1)
                    div_and_gather(j + 1, b1)

                wait_gather(b)

                @pl.when(j >= NBUF)
                def _drain():
                    drain_out(b)

                extract(b)
                fire_out(j, b)
            return 0

        lax.fori_loop(0, steps // NBUF, outer, 0)
        for b in range(NBUF):
            drain_out(b)

    return emb_kernel, num_workers, steps


def kernel(x, weight):
    batch, hist = x.shape
    vocab, dim = weight.shape
    n_flat = batch * hist
    emb, num_workers, steps = _make_kernel(n_flat, vocab)
    idx = x.reshape(num_workers, steps, CHUNK)
    table = weight.reshape(vocab // PACK, dim * PACK)
    out = emb(idx, table)
    return out.reshape(batch, hist, dim)


# speedup vs baseline: 1.1640x; 1.1640x over previous
import functools

import jax
import jax.numpy as jnp
from jax import lax
from jax.experimental import pallas as pl
from jax.experimental.pallas import tpu as pltpu
from jax.experimental.pallas import tpu_sc as plsc

EMBED_DIM = 32
CHUNK = 128
NBUF = 8
PRE = 4


@functools.cache
def _make_kernel(n_flat: int):
    info = plsc.get_sparse_core_info()
    num_workers = info.num_cores * info.num_subcores
    b_per_w = n_flat // num_workers
    steps = b_per_w // CHUNK
    mesh = plsc.VectorSubcoreMesh(core_axis_name="c", subcore_axis_name="s")

    @functools.partial(
        pl.kernel,
        mesh=mesh,
        out_type=jax.ShapeDtypeStruct((n_flat, EMBED_DIM), jnp.float32),
        scratch_types=[
            pltpu.VMEM((steps, CHUNK), jnp.int32),
            *[pltpu.VMEM((CHUNK, EMBED_DIM), jnp.float32) for _ in range(NBUF)],
            pltpu.SemaphoreType.DMA,
            pltpu.SemaphoreType.DMA,
        ],
        compiler_params=pltpu.CompilerParams(
            use_tc_tiling_on_sc=False, needs_layout_passes=False
        ),
    )
    def emb_kernel(idx_hbm, table_hbm, out_hbm, idx_v, *rest):
        bufs = rest[:NBUF]
        gsem = rest[NBUF]
        osem = rest[NBUF + 1]
        wid = lax.axis_index("s") * info.num_cores + lax.axis_index("c")
        base = wid * b_per_w
        pltpu.sync_copy(idx_hbm.at[wid], idx_v)

        for p in range(PRE):
            pltpu.async_copy(table_hbm.at[idx_v.at[p]], bufs[p], gsem)

        def outer(jo, _):
            for b in range(NBUF):
                j = jo * NBUF + b
                jn = j + PRE
                bn = (b + PRE) % NBUF

                @pl.when(jn < steps)
                def _fire():
                    @pl.when(jn >= NBUF)
                    def _drain():
                        pltpu.make_async_copy(
                            bufs[bn], out_hbm.at[pl.ds(base, CHUNK)], osem
                        ).wait()

                    pltpu.async_copy(table_hbm.at[idx_v.at[jn]], bufs[bn], gsem)

                pltpu.make_async_copy(
                    table_hbm.at[idx_v.at[0]], bufs[b], gsem
                ).wait()
                pltpu.async_copy(
                    bufs[b], out_hbm.at[pl.ds(base + j * CHUNK, CHUNK)], osem
                )
            return 0

        lax.fori_loop(0, steps // NBUF, outer, 0)
        for b in range(NBUF):
            pltpu.make_async_copy(
                bufs[b], out_hbm.at[pl.ds(base, CHUNK)], osem
            ).wait()

    return emb_kernel, num_workers, steps


def kernel(x, weight):
    batch, hist = x.shape
    n_flat = batch * hist
    emb, num_workers, steps = _make_kernel(n_flat)
    idx = x.reshape(num_workers, steps, CHUNK)
    out = emb(idx, weight)
    return out.reshape(batch, hist, EMBED_DIM)


# speedup vs baseline: 1.1648x; 1.0007x over previous
import functools

import jax
import jax.numpy as jnp
from jax import lax
from jax.experimental import pallas as pl
from jax.experimental.pallas import tpu as pltpu
from jax.experimental.pallas import tpu_sc as plsc

EMBED_DIM = 32
CHUNK = 128
NBUF = 8
PRE = 4


@functools.cache
def _make_kernel(n_flat: int):
    info = plsc.get_sparse_core_info()
    num_workers = info.num_cores * info.num_subcores
    b_per_w = n_flat // num_workers
    steps = b_per_w // CHUNK
    mesh = plsc.VectorSubcoreMesh(core_axis_name="c", subcore_axis_name="s")

    @functools.partial(
        pl.kernel,
        mesh=mesh,
        out_type=jax.ShapeDtypeStruct((n_flat, EMBED_DIM), jnp.float32),
        scratch_types=[
            pltpu.VMEM((steps, CHUNK), jnp.int32),
            *[pltpu.VMEM((CHUNK, EMBED_DIM), jnp.float32) for _ in range(NBUF)],
            pltpu.SemaphoreType.DMA,
            pltpu.SemaphoreType.DMA,
        ],
        compiler_params=pltpu.CompilerParams(use_tc_tiling_on_sc=False),
    )
    def emb_kernel(idx_hbm, table_hbm, out_hbm, idx_v, *rest):
        bufs = rest[:NBUF]
        gsem = rest[NBUF]
        osem = rest[NBUF + 1]
        wid = lax.axis_index("s") * info.num_cores + lax.axis_index("c")
        base = wid * b_per_w
        pltpu.sync_copy(idx_hbm.at[wid], idx_v)

        for p in range(PRE):
            pltpu.async_copy(table_hbm.at[idx_v.at[p]], bufs[p], gsem)

        def outer(jo, _):
            for b in range(NBUF):
                j = jo * NBUF + b
                jn = j + PRE
                bn = (b + PRE) % NBUF

                @pl.when(jn < steps)
                def _fire():
                    @pl.when(jn >= NBUF)
                    def _drain():
                        pltpu.make_async_copy(
                            bufs[bn], out_hbm.at[pl.ds(base, CHUNK)], osem
                        ).wait()

                    pltpu.async_copy(table_hbm.at[idx_v.at[jn]], bufs[bn], gsem)

                pltpu.make_async_copy(
                    table_hbm.at[idx_v.at[0]], bufs[b], gsem
                ).wait()
                pltpu.async_copy(
                    bufs[b], out_hbm.at[pl.ds(base + j * CHUNK, CHUNK)], osem
                )
            return 0

        lax.fori_loop(0, steps // NBUF, outer, 0)
        for b in range(NBUF):
            pltpu.make_async_copy(
                bufs[b], out_hbm.at[pl.ds(base, CHUNK)], osem
            ).wait()

    return emb_kernel, num_workers, steps


def kernel(x, weight):
    batch, hist = x.shape
    n_flat = batch * hist
    emb, num_workers, steps = _make_kernel(n_flat)
    idx = x.reshape(num_workers, steps, CHUNK)
    out = emb(idx, weight)
    return out.reshape(batch, hist, EMBED_DIM)
